# baseline (speedup 1.0000x reference)
import functools

import jax
import jax.numpy as jnp
from jax import lax
from jax.experimental import pallas as pl
from jax.experimental.pallas import tpu as pltpu
from jax.experimental.pallas import tpu_sc as plsc


_MESH = plsc.VectorSubcoreMesh(core_axis_name="c", subcore_axis_name="s")
_SC_PARAMS = pltpu.CompilerParams(use_tc_tiling_on_sc=False)
_NT = 16
_NPAD = 100096
_RPT = _NPAD // _NT
_WB = 184
_KE = 800
_EPAD = 1638400


def _degree_kernel(dst, n, e):
    epc = e // 2
    ept = epc // _NT
    K = 1000
    nblk = ept // K

    @functools.partial(
        pl.kernel,
        out_type=jax.ShapeDtypeStruct((2, _NPAD, 16), jnp.float32),
        mesh=_MESH,
        compiler_params=_SC_PARAMS,
        scratch_types=[
            pltpu.VMEM((K,), jnp.int32),
            pltpu.VMEM((K, 16), jnp.float32),
            pltpu.VMEM((_WB, 16), jnp.float32),
            pltpu.VMEM_SHARED((_NPAD, 16), jnp.float32),
        ],
    )
    def k(dst_hbm, out_hbm, dst_v, ones_v, wb_v, acc_sh):
        cid = lax.axis_index("c")
        sid = lax.axis_index("s")

        @pl.loop(0, K)
        def _(i):
            ones_v[i, :] = jnp.full((16,), 1.0, jnp.float32)

        @pl.loop(0, _WB)
        def _(i):
            wb_v[i, :] = jnp.zeros((16,), jnp.float32)

        rbase = sid * _RPT

        @pl.loop(0, _RPT // _WB)
        def _(b):
            pltpu.sync_copy(wb_v, acc_sh.at[pl.ds(rbase + b * _WB, _WB)])

        plsc.subcore_barrier()

        ebase = cid * epc + sid * ept

        @pl.loop(0, nblk)
        def _(j):
            pltpu.sync_copy(dst_hbm.at[pl.ds(ebase + j * K, K)], dst_v)
            pltpu.sync_copy(ones_v, acc_sh.at[dst_v], add=True)

        plsc.subcore_barrier()

        @pl.loop(0, _RPT // _WB)
        def _(b):
            r0 = rbase + b * _WB
            pltpu.sync_copy(acc_sh.at[pl.ds(r0, _WB)], wb_v)
            pltpu.sync_copy(wb_v, out_hbm.at[cid, pl.ds(r0, _WB)])

    return k(dst)


def _spmm_kernel(hs_flat, src_scaled, dst, nc, e):
    split_edges = nc == 1
    ept = (e // 2 // _NT) if split_edges else (e // _NT)
    nblk = ept // _KE
    has_tail = nblk % 2 == 1
    npair = (nblk - 1) // 2 if has_tail else nblk // 2
    nc2 = 1 if split_edges else nc // 2
    out_cols = 32 if split_edges else nc * 16

    @functools.partial(
        pl.kernel,
        out_type=jax.ShapeDtypeStruct((_NPAD, out_cols), jnp.float32),
        mesh=_MESH,
        compiler_params=_SC_PARAMS,
        scratch_types=[
            pltpu.VMEM((_KE,), jnp.int32),
            pltpu.VMEM((_KE,), jnp.int32),
            pltpu.VMEM((_KE,), jnp.int32),
            pltpu.VMEM((_KE,), jnp.int32),
            pltpu.VMEM((_KE, 16), jnp.float32),
            pltpu.VMEM((_KE, 16), jnp.float32),
            pltpu.VMEM_SHARED((_NPAD, 16), jnp.float32),
            pltpu.SemaphoreType.DMA,
            pltpu.SemaphoreType.DMA,
            pltpu.SemaphoreType.DMA,
            pltpu.SemaphoreType.DMA,
            pltpu.SemaphoreType.DMA,
            pltpu.SemaphoreType.DMA,
            pltpu.SemaphoreType.DMA,
            pltpu.SemaphoreType.DMA,
        ],
    )
    def k(hs_hbm, src_hbm, dst_hbm, out_hbm,
          src0, src1, dst0, dst1, rows0, rows1, acc_sh,
          ssem0, ssem1, dsem0, dsem1, gsem0, gsem1, csem0, csem1):
        cid = lax.axis_index("c")
        sid = lax.axis_index("s")
        srcb = (src0, src1)
        dstb = (dst0, dst1)
        rowsb = (rows0, rows1)
        ssem = (ssem0, ssem1)
        dsem = (dsem0, dsem1)
        gsem = (gsem0, gsem1)
        csem = (csem0, csem1)
        rbase = sid * _RPT
        if split_edges:
            ebase = cid * (e // 2) + sid * ept
        else:
            ebase = sid * ept

        def start_idx(b, p, chunk):
            base = ebase + b * _KE
            gbase = base if split_edges else chunk * e + base
            pltpu.async_copy(src_hbm.at[pl.ds(gbase, _KE)], srcb[p], ssem[p])
            pltpu.async_copy(dst_hbm.at[pl.ds(base, _KE)], dstb[p], dsem[p])

        def wait_idx(b, p, chunk):
            base = ebase + b * _KE
            gbase = base if split_edges else chunk * e + base
            pltpu.make_async_copy(
                src_hbm.at[pl.ds(gbase, _KE)], srcb[p], ssem[p]).wait()
            pltpu.make_async_copy(
                dst_hbm.at[pl.ds(base, _KE)], dstb[p], dsem[p]).wait()

        def wait_scat(p):
            pltpu.make_async_copy(
                rowsb[p], acc_sh.at[dstb[p]], csem[p]).wait()

        def sub_step(b, p, chunk, prefetch_b, wait_scat_cond, prefetch_cond):
            q = 1 - p
            wait_idx(b, p, chunk)

            pltpu.async_copy(hs_hbm.at[srcb[p]], rowsb[p], gsem[p])
            if wait_scat_cond is not None:
                @pl.when(wait_scat_cond)
                def _():
                    wait_scat(q)
            else:
                wait_scat(q)
            if prefetch_cond is not None:
                @pl.when(prefetch_cond)
                def _():
                    start_idx(prefetch_b, q, chunk)
            elif prefetch_b is not None:
                start_idx(prefetch_b, q, chunk)
            pltpu.make_async_copy(hs_hbm.at[srcb[p]], rowsb[p], gsem[p]).wait()
            pltpu.async_copy(rowsb[p], acc_sh.at[dstb[p]], csem[p], add=True)

        @pl.loop(0, _WB)
        def _(i):
            rows0[i, :] = jnp.zeros((16,), jnp.float32)

        @pl.loop(0, _RPT // _WB)
        def _(b):
            pltpu.sync_copy(rows0.at[pl.ds(0, _WB)],
                            acc_sh.at[pl.ds(rbase + b * _WB, _WB)])

        plsc.subcore_barrier()

        @pl.loop(0, nc2)
        def _(cc):
            chunk = cid if split_edges else cid * nc2 + cc
            start_idx(0, 0, chunk)
            start_idx(1, 1, chunk)

            @pl.loop(0, npair)
            def _(j):
                sub_step(2 * j, 0, chunk, 2 * j + 1, j > 0, j > 0)
                if has_tail:
                    sub_step(2 * j + 1, 1, chunk, 2 * j + 2, None, None)
                else:
                    sub_step(2 * j + 1, 1, chunk, 2 * j + 2, None,
                             j < npair - 1)

            if has_tail:
                sub_step(nblk - 1, 0, chunk, None, None, None)
                wait_scat(0)
            else:
                wait_scat(1)

            plsc.subcore_barrier()

            @pl.loop(0, _WB)
            def _(i):
                rows1[i, :] = jnp.zeros((16,), jnp.float32)

            @pl.loop(0, _RPT // _WB)
            def _(b):
                r0 = rbase + b * _WB
                pltpu.sync_copy(acc_sh.at[pl.ds(r0, _WB)],
                                rows0.at[pl.ds(0, _WB)])
                pltpu.sync_copy(
                    rows0.at[pl.ds(0, _WB)],
                    out_hbm.at[pl.ds(r0, _WB), pl.ds(chunk * 16, 16)])
                pltpu.sync_copy(rows1.at[pl.ds(0, _WB)],
                                acc_sh.at[pl.ds(r0, _WB)])

            plsc.subcore_barrier()

    return k(hs_flat, src_scaled, dst)


_BR = 2000


def _scale0_body(x_ref, c0_ref, c1_ref, p_ref, nrm_ref):
    deg = c0_ref[0] + c1_ref[0] + 1.0
    nrm = lax.rsqrt(deg)
    nrm_ref[...] = nrm
    p_ref[...] = x_ref[...] * nrm[:, :1]


def _scale0(x16, counts, n):
    g = n // _BR
    return pl.pallas_call(
        _scale0_body,
        grid=(g,),
        in_specs=[
            pl.BlockSpec((_BR, 16), lambda i: (i, 0)),
            pl.BlockSpec((1, _BR, 16), lambda i: (0, i, 0)),
            pl.BlockSpec((1, _BR, 16), lambda i: (1, i, 0)),
        ],
        out_specs=[
            pl.BlockSpec((_BR, 16), lambda i: (i, 0)),
            pl.BlockSpec((_BR, 16), lambda i: (i, 0)),
        ],
        out_shape=[
            jax.ShapeDtypeStruct((n, 16), jnp.float32),
            jax.ShapeDtypeStruct((n, 16), jnp.float32),
        ],
    )(x16, counts, counts)


def _layer1_body(a_ref, p_ref, nrm_ref, b_ref, w_ref, o_ref):
    nrm = nrm_ref[:, :1]
    z = (a_ref[:, :16] + a_ref[:, 16:] + p_ref[...]) * nrm
    h = jnp.dot(z, w_ref[...], preferred_element_type=jnp.float32) + b_ref[...]
    o_ref[...] = jnp.maximum(h, 0.0) * nrm


def _layer1(accp, p, nrm, b1, w1p, n):
    g = n // _BR
    return pl.pallas_call(
        _layer1_body,
        grid=(g,),
        in_specs=[
            pl.BlockSpec((_BR, 32), lambda i: (i, 0)),
            pl.BlockSpec((_BR, 16), lambda i: (i, 0)),
            pl.BlockSpec((_BR, 16), lambda i: (i, 0)),
            pl.BlockSpec((1, 128), lambda i: (0, 0)),
            pl.BlockSpec((16, 128), lambda i: (0, 0)),
        ],
        out_specs=pl.BlockSpec((_BR, 128), lambda i: (i, 0)),
        out_shape=jax.ShapeDtypeStruct((n, 128), jnp.float32),
    )(accp, p, nrm, b1, w1p)


def _layer2_body(a_ref, q_ref, nrm_ref, b_ref, w_ref, o_ref):
    nrm = nrm_ref[:, :1]
    z = (a_ref[...] + q_ref[...]) * nrm
    h = jnp.dot(z, w_ref[...], preferred_element_type=jnp.float32) + b_ref[...]
    o_ref[...] = jnp.maximum(h, 0.0) * nrm


def _layer2(acc, q, nrm, b_prev, w, n):
    g = n // _BR
    f = q.shape[1]
    f2 = w.shape[1]
    return pl.pallas_call(
        _layer2_body,
        grid=(g,),
        in_specs=[
            pl.BlockSpec((_BR, f), lambda i: (i, 0)),
            pl.BlockSpec((_BR, f), lambda i: (i, 0)),
            pl.BlockSpec((_BR, 16), lambda i: (i, 0)),
            pl.BlockSpec((1, f2), lambda i: (0, 0)),
            pl.BlockSpec((f, f2), lambda i: (0, 0)),
        ],
        out_specs=pl.BlockSpec((_BR, f2), lambda i: (i, 0)),
        out_shape=jax.ShapeDtypeStruct((n, f2), jnp.float32),
    )(acc, q, nrm, b_prev, w)


def _head_body(acc_ref, q_ref, nrm_ref, b_ref, w_ref, bat_ref,
               wmu_ref, bmu_ref, wlv_ref, blv_ref,
               mu_ref, lv_ref, psum, cnt, ng):
    i = pl.program_id(0)

    @pl.when(i == 0)
    def _():
        psum[...] = jnp.zeros_like(psum)
        cnt[...] = jnp.zeros_like(cnt)

    nrm = nrm_ref[:, :1]
    z = (acc_ref[...] + q_ref[...]) * nrm
    h = jnp.dot(z, w_ref[...], preferred_element_type=jnp.float32) + b_ref[...]
    h = jnp.maximum(h, 0.0)
    bb = bat_ref[:, :1]
    io = lax.broadcasted_iota(jnp.int32, (_BR, 256), 1)
    oh = jnp.where(bb == io, 1.0, 0.0)
    psum[...] += lax.dot_general(oh, h, (((0,), (0,)), ((), ())),
                                 preferred_element_type=jnp.float32)
    cnt[...] += lax.dot_general(oh, jnp.ones((_BR, 8), jnp.float32),
                                (((0,), (0,)), ((), ())),
                                preferred_element_type=jnp.float32)

    @pl.when(i == ng - 1)
    def _():
        pooled = psum[...] / jnp.maximum(cnt[:, :1], 1.0)
        mu_ref[...] = jnp.dot(pooled, wmu_ref[...],
                              preferred_element_type=jnp.float32) + bmu_ref[...]
        lv_ref[...] = jnp.dot(pooled, wlv_ref[...],
                              preferred_element_type=jnp.float32) + blv_ref[...]


def _head(acc, q, nrm, b3, w3, batchi, wmu, bmu, wlv, blv, n):
    g = n // _BR
    return pl.pallas_call(
        functools.partial(_head_body, ng=g),
        grid=(g,),
        in_specs=[
            pl.BlockSpec((_BR, 256), lambda i: (i, 0)),
            pl.BlockSpec((_BR, 256), lambda i: (i, 0)),
            pl.BlockSpec((_BR, 16), lambda i: (i, 0)),
            pl.BlockSpec((1, 256), lambda i: (0, 0)),
            pl.BlockSpec((256, 256), lambda i: (0, 0)),
            pl.BlockSpec((_BR, 8), lambda i: (i, 0)),
            pl.BlockSpec((256, 64), lambda i: (0, 0)),
            pl.BlockSpec((1, 64), lambda i: (0, 0)),
            pl.BlockSpec((256, 64), lambda i: (0, 0)),
            pl.BlockSpec((1, 64), lambda i: (0, 0)),
        ],
        out_specs=[
            pl.BlockSpec((256, 64), lambda i: (0, 0)),
            pl.BlockSpec((256, 64), lambda i: (0, 0)),
        ],
        out_shape=[
            jax.ShapeDtypeStruct((256, 64), jnp.float32),
            jax.ShapeDtypeStruct((256, 64), jnp.float32),
        ],
        scratch_shapes=[
            pltpu.VMEM((256, 256), jnp.float32),
            pltpu.VMEM((256, 8), jnp.float32),
        ],
    )(acc, q, nrm, b3, w3, batchi, wmu, bmu, wlv, blv)


def kernel(x, edge_index, batch, W1, b1, W2, b2, W3, b3, Wmu, bmu, Wlv, blv):
    n = x.shape[0]
    e = edge_index.shape[1]
    src = edge_index[0]
    dst = edge_index[1]

    counts = _degree_kernel(dst, n, e)

    x16 = jnp.pad(x, ((0, 0), (0, 6)))
    p, nrm = _scale0(x16, counts, n)

    npads = _EPAD - e
    fill = (jnp.arange(npads, dtype=jnp.int32) % 96) + n
    src1p = jnp.concatenate([src, fill])
    dst1p = jnp.concatenate([dst, fill])
    p_pad = jnp.pad(p, ((0, 96), (0, 0)))
    accp = _spmm_kernel(p_pad, src1p, dst1p, 1, _EPAD)

    w1p = jnp.pad(W1, ((0, 6), (0, 0)))
    q1 = _layer1(accp, p, nrm, b1.reshape(1, 128), w1p, n)

    gidx8 = (src[None, :] * 8 + jnp.arange(8, dtype=jnp.int32)[:, None]
             ).reshape(8 * e)
    gidx16 = (src[None, :] * 16 + jnp.arange(16, dtype=jnp.int32)[:, None]
              ).reshape(16 * e)
    acc1 = _spmm_kernel(q1.reshape(n * 8, 16), gidx8, dst, 8, e)
    q2 = _layer2(acc1, q1, nrm, b2.reshape(1, 256), W2, n)

    acc2 = _spmm_kernel(q2.reshape(n * 16, 16), gidx16, dst, 16, e)

    batchi = jnp.broadcast_to(batch[:, None], (n, 8))
    mu, lv = _head(acc2, q2, nrm, b3.reshape(1, 256), W3,
                   batchi, Wmu, bmu.reshape(1, 64), Wlv, blv.reshape(1, 64), n)
    return (mu, lv)


# speedup vs baseline: 1.5798x; 1.5798x over previous
import functools

import jax
import jax.numpy as jnp
from jax import lax
from jax.experimental import pallas as pl
from jax.experimental.pallas import tpu as pltpu
from jax.experimental.pallas import tpu_sc as plsc


_MESH = plsc.VectorSubcoreMesh(core_axis_name="c", subcore_axis_name="s")
_SC_PARAMS = pltpu.CompilerParams(use_tc_tiling_on_sc=False)
_NT = 16
_NPAD = 100096
_RPT = _NPAD // _NT
_WB = 184
_KE = 800
_EPAD = 1638400


def _degree_kernel(dst, n, e):
    epc = e // 2
    ept = epc // _NT
    K = 1000
    nblk = ept // K

    @functools.partial(
        pl.kernel,
        out_type=jax.ShapeDtypeStruct((2, _NPAD, 16), jnp.float32),
        mesh=_MESH,
        compiler_params=_SC_PARAMS,
        scratch_types=[
            pltpu.VMEM((K,), jnp.int32),
            pltpu.VMEM((K, 16), jnp.float32),
            pltpu.VMEM((_WB, 16), jnp.float32),
            pltpu.VMEM_SHARED((_NPAD, 16), jnp.float32),
        ],
    )
    def k(dst_hbm, out_hbm, dst_v, ones_v, wb_v, acc_sh):
        cid = lax.axis_index("c")
        sid = lax.axis_index("s")

        @pl.loop(0, K)
        def _(i):
            ones_v[i, :] = jnp.full((16,), 1.0, jnp.float32)

        @pl.loop(0, _WB)
        def _(i):
            wb_v[i, :] = jnp.zeros((16,), jnp.float32)

        rbase = sid * _RPT

        @pl.loop(0, _RPT // _WB)
        def _(b):
            pltpu.sync_copy(wb_v, acc_sh.at[pl.ds(rbase + b * _WB, _WB)])

        plsc.subcore_barrier()

        ebase = cid * epc + sid * ept

        @pl.loop(0, nblk)
        def _(j):
            pltpu.sync_copy(dst_hbm.at[pl.ds(ebase + j * K, K)], dst_v)
            pltpu.sync_copy(ones_v, acc_sh.at[dst_v], add=True)

        plsc.subcore_barrier()

        @pl.loop(0, _RPT // _WB)
        def _(b):
            r0 = rbase + b * _WB
            pltpu.sync_copy(acc_sh.at[pl.ds(r0, _WB)], wb_v)
            pltpu.sync_copy(wb_v, out_hbm.at[cid, pl.ds(r0, _WB)])

    return k(dst)


def _spmm_kernel(hs_flat, src_scaled, dst, nc, e):
    split_edges = nc == 1
    ept = (e // 2 // _NT) if split_edges else (e // _NT)
    nblk = ept // _KE
    has_tail = nblk % 2 == 1
    npair = (nblk - 1) // 2 if has_tail else nblk // 2
    nc2 = 1 if split_edges else nc // 2
    out_cols = 32 if split_edges else nc * 16

    @functools.partial(
        pl.kernel,
        out_type=jax.ShapeDtypeStruct((_NPAD, out_cols), jnp.float32),
        mesh=_MESH,
        compiler_params=_SC_PARAMS,
        scratch_types=[
            pltpu.VMEM((_KE,), jnp.int32),
            pltpu.VMEM((_KE,), jnp.int32),
            pltpu.VMEM((_KE,), jnp.int32),
            pltpu.VMEM((_KE,), jnp.int32),
            pltpu.VMEM((_KE, 16), jnp.float32),
            pltpu.VMEM((_KE, 16), jnp.float32),
            pltpu.VMEM_SHARED((_NPAD, 16), jnp.float32),
            pltpu.SemaphoreType.DMA,
            pltpu.SemaphoreType.DMA,
            pltpu.SemaphoreType.DMA,
            pltpu.SemaphoreType.DMA,
            pltpu.SemaphoreType.DMA,
            pltpu.SemaphoreType.DMA,
            pltpu.SemaphoreType.DMA,
            pltpu.SemaphoreType.DMA,
        ],
    )
    def k(hs_hbm, src_hbm, dst_hbm, out_hbm,
          src0, src1, dst0, dst1, rows0, rows1, acc_sh,
          ssem0, ssem1, dsem0, dsem1, gsem0, gsem1, csem0, csem1):
        cid = lax.axis_index("c")
        sid = lax.axis_index("s")
        srcb = (src0, src1)
        dstb = (dst0, dst1)
        rowsb = (rows0, rows1)
        ssem = (ssem0, ssem1)
        dsem = (dsem0, dsem1)
        gsem = (gsem0, gsem1)
        csem = (csem0, csem1)
        rbase = sid * _RPT
        if split_edges:
            ebase = cid * (e // 2) + sid * ept
        else:
            ebase = sid * ept

        def start_idx(b, p, chunk):
            base = ebase + b * _KE
            pltpu.async_copy(src_hbm.at[pl.ds(base, _KE)], srcb[p], ssem[p])
            pltpu.async_copy(dst_hbm.at[pl.ds(base, _KE)], dstb[p], dsem[p])

        def wait_idx(b, p, chunk):
            base = ebase + b * _KE
            pltpu.make_async_copy(
                src_hbm.at[pl.ds(base, _KE)], srcb[p], ssem[p]).wait()
            pltpu.make_async_copy(
                dst_hbm.at[pl.ds(base, _KE)], dstb[p], dsem[p]).wait()

        def wait_scat(p):
            pltpu.make_async_copy(
                rowsb[p], acc_sh.at[dstb[p]], csem[p]).wait()

        def sub_step(b, p, chunk, prefetch_b, wait_scat_cond, prefetch_cond):
            q = 1 - p
            wait_idx(b, p, chunk)

            if not split_edges:
                @pl.loop(0, _KE // 16)
                def _(i):
                    s = srcb[p][pl.ds(i * 16, 16)]
                    srcb[p][pl.ds(i * 16, 16)] = s + chunk

            pltpu.async_copy(hs_hbm.at[srcb[p]], rowsb[p], gsem[p])
            if wait_scat_cond is not None:
                @pl.when(wait_scat_cond)
                def _():
                    wait_scat(q)
            else:
                wait_scat(q)
            if prefetch_cond is not None:
                @pl.when(prefetch_cond)
                def _():
                    start_idx(prefetch_b, q, chunk)
            elif prefetch_b is not None:
                start_idx(prefetch_b, q, chunk)
            pltpu.make_async_copy(hs_hbm.at[srcb[p]], rowsb[p], gsem[p]).wait()
            pltpu.async_copy(rowsb[p], acc_sh.at[dstb[p]], csem[p], add=True)

        @pl.loop(0, _WB)
        def _(i):
            rows0[i, :] = jnp.zeros((16,), jnp.float32)

        @pl.loop(0, _RPT // _WB)
        def _(b):
            pltpu.sync_copy(rows0.at[pl.ds(0, _WB)],
                            acc_sh.at[pl.ds(rbase + b * _WB, _WB)])

        plsc.subcore_barrier()

        @pl.loop(0, nc2)
        def _(cc):
            chunk = cid if split_edges else cid * nc2 + cc
            start_idx(0, 0, chunk)
            start_idx(1, 1, chunk)

            @pl.loop(0, npair)
            def _(j):
                sub_step(2 * j, 0, chunk, 2 * j + 1, j > 0, j > 0)
                if has_tail:
                    sub_step(2 * j + 1, 1, chunk, 2 * j + 2, None, None)
                else:
                    sub_step(2 * j + 1, 1, chunk, 2 * j + 2, None,
                             j < npair - 1)

            if has_tail:
                sub_step(nblk - 1, 0, chunk, None, None, None)
                wait_scat(0)
            else:
                wait_scat(1)

            plsc.subcore_barrier()

            @pl.loop(0, _WB)
            def _(i):
                rows1[i, :] = jnp.zeros((16,), jnp.float32)

            @pl.loop(0, _RPT // _WB)
            def _(b):
                r0 = rbase + b * _WB
                pltpu.sync_copy(acc_sh.at[pl.ds(r0, _WB)],
                                rows0.at[pl.ds(0, _WB)])
                pltpu.sync_copy(
                    rows0.at[pl.ds(0, _WB)],
                    out_hbm.at[pl.ds(r0, _WB), pl.ds(chunk * 16, 16)])
                pltpu.sync_copy(rows1.at[pl.ds(0, _WB)],
                                acc_sh.at[pl.ds(r0, _WB)])

            plsc.subcore_barrier()

    return k(hs_flat, src_scaled, dst)


_BR = 2000


def _scale0_body(x_ref, c0_ref, c1_ref, p_ref, nrm_ref):
    deg = c0_ref[0] + c1_ref[0] + 1.0
    nrm = lax.rsqrt(deg)
    nrm_ref[...] = nrm
    p_ref[...] = x_ref[...] * nrm[:, :1]


def _scale0(x16, counts, n):
    g = n // _BR
    return pl.pallas_call(
        _scale0_body,
        grid=(g,),
        in_specs=[
            pl.BlockSpec((_BR, 16), lambda i: (i, 0)),
            pl.BlockSpec((1, _BR, 16), lambda i: (0, i, 0)),
            pl.BlockSpec((1, _BR, 16), lambda i: (1, i, 0)),
        ],
        out_specs=[
            pl.BlockSpec((_BR, 16), lambda i: (i, 0)),
            pl.BlockSpec((_BR, 16), lambda i: (i, 0)),
        ],
        out_shape=[
            jax.ShapeDtypeStruct((n, 16), jnp.float32),
            jax.ShapeDtypeStruct((n, 16), jnp.float32),
        ],
    )(x16, counts, counts)


def _layer1_body(a_ref, p_ref, nrm_ref, b_ref, w_ref, o_ref):
    nrm = nrm_ref[:, :1]
    z = (a_ref[:, :16] + a_ref[:, 16:] + p_ref[...]) * nrm
    h = jnp.dot(z, w_ref[...], preferred_element_type=jnp.float32) + b_ref[...]
    o_ref[...] = jnp.maximum(h, 0.0) * nrm


def _layer1(accp, p, nrm, b1, w1p, n):
    g = n // _BR
    return pl.pallas_call(
        _layer1_body,
        grid=(g,),
        in_specs=[
            pl.BlockSpec((_BR, 32), lambda i: (i, 0)),
            pl.BlockSpec((_BR, 16), lambda i: (i, 0)),
            pl.BlockSpec((_BR, 16), lambda i: (i, 0)),
            pl.BlockSpec((1, 128), lambda i: (0, 0)),
            pl.BlockSpec((16, 128), lambda i: (0, 0)),
        ],
        out_specs=pl.BlockSpec((_BR, 128), lambda i: (i, 0)),
        out_shape=jax.ShapeDtypeStruct((n, 128), jnp.float32),
    )(accp, p, nrm, b1, w1p)


def _layer2_body(a_ref, q_ref, nrm_ref, b_ref, w_ref, o_ref):
    nrm = nrm_ref[:, :1]
    z = (a_ref[...] + q_ref[...]) * nrm
    h = jnp.dot(z, w_ref[...], preferred_element_type=jnp.float32) + b_ref[...]
    o_ref[...] = jnp.maximum(h, 0.0) * nrm


def _layer2(acc, q, nrm, b_prev, w, n):
    g = n // _BR
    f = q.shape[1]
    f2 = w.shape[1]
    return pl.pallas_call(
        _layer2_body,
        grid=(g,),
        in_specs=[
            pl.BlockSpec((_BR, f), lambda i: (i, 0)),
            pl.BlockSpec((_BR, f), lambda i: (i, 0)),
            pl.BlockSpec((_BR, 16), lambda i: (i, 0)),
            pl.BlockSpec((1, f2), lambda i: (0, 0)),
            pl.BlockSpec((f, f2), lambda i: (0, 0)),
        ],
        out_specs=pl.BlockSpec((_BR, f2), lambda i: (i, 0)),
        out_shape=jax.ShapeDtypeStruct((n, f2), jnp.float32),
    )(acc, q, nrm, b_prev, w)


def _head_body(acc_ref, q_ref, nrm_ref, b_ref, w_ref, bat_ref,
               wmu_ref, bmu_ref, wlv_ref, blv_ref,
               mu_ref, lv_ref, psum, cnt, ng):
    i = pl.program_id(0)

    @pl.when(i == 0)
    def _():
        psum[...] = jnp.zeros_like(psum)
        cnt[...] = jnp.zeros_like(cnt)

    nrm = nrm_ref[:, :1]
    z = (acc_ref[...] + q_ref[...]) * nrm
    h = jnp.dot(z, w_ref[...], preferred_element_type=jnp.float32) + b_ref[...]
    h = jnp.maximum(h, 0.0)
    bb = bat_ref[:, :1]
    io = lax.broadcasted_iota(jnp.int32, (_BR, 256), 1)
    oh = jnp.where(bb == io, 1.0, 0.0)
    psum[...] += lax.dot_general(oh, h, (((0,), (0,)), ((), ())),
                                 preferred_element_type=jnp.float32)
    cnt[...] += lax.dot_general(oh, jnp.ones((_BR, 8), jnp.float32),
                                (((0,), (0,)), ((), ())),
                                preferred_element_type=jnp.float32)

    @pl.when(i == ng - 1)
    def _():
        pooled = psum[...] / jnp.maximum(cnt[:, :1], 1.0)
        mu_ref[...] = jnp.dot(pooled, wmu_ref[...],
                              preferred_element_type=jnp.float32) + bmu_ref[...]
        lv_ref[...] = jnp.dot(pooled, wlv_ref[...],
                              preferred_element_type=jnp.float32) + blv_ref[...]


def _head(acc, q, nrm, b3, w3, batchi, wmu, bmu, wlv, blv, n):
    g = n // _BR
    return pl.pallas_call(
        functools.partial(_head_body, ng=g),
        grid=(g,),
        in_specs=[
            pl.BlockSpec((_BR, 256), lambda i: (i, 0)),
            pl.BlockSpec((_BR, 256), lambda i: (i, 0)),
            pl.BlockSpec((_BR, 16), lambda i: (i, 0)),
            pl.BlockSpec((1, 256), lambda i: (0, 0)),
            pl.BlockSpec((256, 256), lambda i: (0, 0)),
            pl.BlockSpec((_BR, 8), lambda i: (i, 0)),
            pl.BlockSpec((256, 64), lambda i: (0, 0)),
            pl.BlockSpec((1, 64), lambda i: (0, 0)),
            pl.BlockSpec((256, 64), lambda i: (0, 0)),
            pl.BlockSpec((1, 64), lambda i: (0, 0)),
        ],
        out_specs=[
            pl.BlockSpec((256, 64), lambda i: (0, 0)),
            pl.BlockSpec((256, 64), lambda i: (0, 0)),
        ],
        out_shape=[
            jax.ShapeDtypeStruct((256, 64), jnp.float32),
            jax.ShapeDtypeStruct((256, 64), jnp.float32),
        ],
        scratch_shapes=[
            pltpu.VMEM((256, 256), jnp.float32),
            pltpu.VMEM((256, 8), jnp.float32),
        ],
    )(acc, q, nrm, b3, w3, batchi, wmu, bmu, wlv, blv)


def kernel(x, edge_index, batch, W1, b1, W2, b2, W3, b3, Wmu, bmu, Wlv, blv):
    n = x.shape[0]
    e = edge_index.shape[1]
    src = edge_index[0]
    dst = edge_index[1]

    counts = _degree_kernel(dst, n, e)

    x16 = jnp.pad(x, ((0, 0), (0, 6)))
    p, nrm = _scale0(x16, counts, n)

    npads = _EPAD - e
    fill = (jnp.arange(npads, dtype=jnp.int32) % 96) + n
    src1p = jnp.concatenate([src, fill])
    dst1p = jnp.concatenate([dst, fill])
    p_pad = jnp.pad(p, ((0, 96), (0, 0)))
    accp = _spmm_kernel(p_pad, src1p, dst1p, 1, _EPAD)

    w1p = jnp.pad(W1, ((0, 6), (0, 0)))
    q1 = _layer1(accp, p, nrm, b1.reshape(1, 128), w1p, n)

    acc1 = _spmm_kernel(q1.reshape(n * 8, 16), src * 8, dst, 8, e)
    q2 = _layer2(acc1, q1, nrm, b2.reshape(1, 256), W2, n)

    acc2 = _spmm_kernel(q2.reshape(n * 16, 16), src * 16, dst, 16, e)

    batchi = jnp.broadcast_to(batch[:, None], (n, 8))
    mu, lv = _head(acc2, q2, nrm, b3.reshape(1, 256), W3,
                   batchi, Wmu, bmu.reshape(1, 64), Wlv, blv.reshape(1, 64), n)
    return (mu, lv)


# speedup vs baseline: 1.6269x; 1.0298x over previous
import functools

import jax
import jax.numpy as jnp
from jax import lax
from jax.experimental import pallas as pl
from jax.experimental.pallas import tpu as pltpu
from jax.experimental.pallas import tpu_sc as plsc


_MESH = plsc.VectorSubcoreMesh(core_axis_name="c", subcore_axis_name="s")
_SC_PARAMS = pltpu.CompilerParams(use_tc_tiling_on_sc=False)
_NT = 16
_NPAD = 100096
_RPT = _NPAD // _NT
_WB = 368
_KE = 800
_EPAD = 1638400


def _degree_kernel(dst, n, e):
    epc = e // 2
    ept = epc // _NT
    K = 1000
    nblk = ept // K

    @functools.partial(
        pl.kernel,
        out_type=jax.ShapeDtypeStruct((2, _NPAD, 16), jnp.float32),
        mesh=_MESH,
        compiler_params=_SC_PARAMS,
        scratch_types=[
            pltpu.VMEM((K,), jnp.int32),
            pltpu.VMEM((K, 16), jnp.float32),
            pltpu.VMEM((_WB, 16), jnp.float32),
            pltpu.VMEM_SHARED((_NPAD, 16), jnp.float32),
        ],
    )
    def k(dst_hbm, out_hbm, dst_v, ones_v, wb_v, acc_sh):
        cid = lax.axis_index("c")
        sid = lax.axis_index("s")

        @pl.loop(0, K)
        def _(i):
            ones_v[i, :] = jnp.full((16,), 1.0, jnp.float32)

        @pl.loop(0, _WB)
        def _(i):
            wb_v[i, :] = jnp.zeros((16,), jnp.float32)

        rbase = sid * _RPT

        @pl.loop(0, _RPT // _WB)
        def _(b):
            pltpu.sync_copy(wb_v, acc_sh.at[pl.ds(rbase + b * _WB, _WB)])

        plsc.subcore_barrier()

        ebase = cid * epc + sid * ept

        @pl.loop(0, nblk)
        def _(j):
            pltpu.sync_copy(dst_hbm.at[pl.ds(ebase + j * K, K)], dst_v)
            pltpu.sync_copy(ones_v, acc_sh.at[dst_v], add=True)

        plsc.subcore_barrier()

        @pl.loop(0, _RPT // _WB)
        def _(b):
            r0 = rbase + b * _WB
            pltpu.sync_copy(acc_sh.at[pl.ds(r0, _WB)], wb_v)
            pltpu.sync_copy(wb_v, out_hbm.at[cid, pl.ds(r0, _WB)])

    return k(dst)


def _spmm_kernel(hs_flat, src_scaled, dst, nc, e):
    split_edges = nc == 1
    ept = (e // 2 // _NT) if split_edges else (e // _NT)
    nblk = ept // _KE
    has_tail = nblk % 2 == 1
    npair = (nblk - 1) // 2 if has_tail else nblk // 2
    nc2 = 1 if split_edges else nc // 2
    out_cols = 32 if split_edges else nc * 16

    @functools.partial(
        pl.kernel,
        out_type=jax.ShapeDtypeStruct((_NPAD, out_cols), jnp.float32),
        mesh=_MESH,
        compiler_params=_SC_PARAMS,
        scratch_types=[
            pltpu.VMEM((_KE,), jnp.int32),
            pltpu.VMEM((_KE,), jnp.int32),
            pltpu.VMEM((_KE,), jnp.int32),
            pltpu.VMEM((_KE,), jnp.int32),
            pltpu.VMEM((_KE, 16), jnp.float32),
            pltpu.VMEM((_KE, 16), jnp.float32),
            pltpu.VMEM_SHARED((_NPAD, 16), jnp.float32),
            pltpu.SemaphoreType.DMA,
            pltpu.SemaphoreType.DMA,
            pltpu.SemaphoreType.DMA,
            pltpu.SemaphoreType.DMA,
            pltpu.SemaphoreType.DMA,
            pltpu.SemaphoreType.DMA,
            pltpu.SemaphoreType.DMA,
            pltpu.SemaphoreType.DMA,
        ],
    )
    def k(hs_hbm, src_hbm, dst_hbm, out_hbm,
          src0, src1, dst0, dst1, rows0, rows1, acc_sh,
          ssem0, ssem1, dsem0, dsem1, gsem0, gsem1, csem0, csem1):
        cid = lax.axis_index("c")
        sid = lax.axis_index("s")
        srcb = (src0, src1)
        dstb = (dst0, dst1)
        rowsb = (rows0, rows1)
        ssem = (ssem0, ssem1)
        dsem = (dsem0, dsem1)
        gsem = (gsem0, gsem1)
        csem = (csem0, csem1)
        rbase = sid * _RPT
        if split_edges:
            ebase = cid * (e // 2) + sid * ept
        else:
            ebase = sid * ept

        def start_idx(b, p, chunk):
            base = ebase + b * _KE
            pltpu.async_copy(src_hbm.at[pl.ds(base, _KE)], srcb[p], ssem[p])
            pltpu.async_copy(dst_hbm.at[pl.ds(base, _KE)], dstb[p], dsem[p])

        def wait_idx(b, p, chunk):
            base = ebase + b * _KE
            pltpu.make_async_copy(
                src_hbm.at[pl.ds(base, _KE)], srcb[p], ssem[p]).wait()
            pltpu.make_async_copy(
                dst_hbm.at[pl.ds(base, _KE)], dstb[p], dsem[p]).wait()

        def wait_scat(p):
            pltpu.make_async_copy(
                rowsb[p], acc_sh.at[dstb[p]], csem[p]).wait()

        def sub_step(b, p, chunk, prefetch_b, wait_scat_cond, prefetch_cond):
            q = 1 - p
            wait_idx(b, p, chunk)

            if not split_edges:
                @pl.loop(0, _KE // 16)
                def _(i):
                    s = srcb[p][pl.ds(i * 16, 16)]
                    srcb[p][pl.ds(i * 16, 16)] = s + chunk

            pltpu.async_copy(hs_hbm.at[srcb[p]], rowsb[p], gsem[p])
            if wait_scat_cond is not None:
                @pl.when(wait_scat_cond)
                def _():
                    wait_scat(q)
            else:
                wait_scat(q)
            if prefetch_cond is not None:
                @pl.when(prefetch_cond)
                def _():
                    start_idx(prefetch_b, q, chunk)
            elif prefetch_b is not None:
                start_idx(prefetch_b, q, chunk)
            pltpu.make_async_copy(hs_hbm.at[srcb[p]], rowsb[p], gsem[p]).wait()
            pltpu.async_copy(rowsb[p], acc_sh.at[dstb[p]], csem[p], add=True)

        @pl.loop(0, _WB)
        def _(i):
            rows1[i, :] = jnp.zeros((16,), jnp.float32)

        @pl.loop(0, _RPT // _WB)
        def _(b):
            pltpu.sync_copy(rows1.at[pl.ds(0, _WB)],
                            acc_sh.at[pl.ds(rbase + b * _WB, _WB)])

        plsc.subcore_barrier()

        @pl.loop(0, nc2)
        def _(cc):
            chunk = cid if split_edges else cid * nc2 + cc
            start_idx(0, 0, chunk)
            start_idx(1, 1, chunk)

            @pl.loop(0, npair)
            def _(j):
                sub_step(2 * j, 0, chunk, 2 * j + 1, j > 0, j > 0)
                if has_tail:
                    sub_step(2 * j + 1, 1, chunk, 2 * j + 2, None, None)
                else:
                    sub_step(2 * j + 1, 1, chunk, 2 * j + 2, None,
                             j < npair - 1)

            if has_tail:
                sub_step(nblk - 1, 0, chunk, None, None, None)
                wait_scat(0)
            else:
                wait_scat(1)

            plsc.subcore_barrier()

            @pl.loop(0, _WB)
            def _(i):
                rows1[i, :] = jnp.zeros((16,), jnp.float32)

            def wb_wait(b, half):
                r0 = rbase + b * _WB
                pltpu.make_async_copy(
                    rows0.at[pl.ds(half * _WB, _WB)],
                    out_hbm.at[pl.ds(r0, _WB), pl.ds(chunk * 16, 16)],
                    gsem[half]).wait()

            def wb_sub(b, half, wait_b, wait_cond):
                r0 = rbase + b * _WB
                if wait_cond is not None:
                    @pl.when(wait_cond)
                    def _():
                        wb_wait(wait_b, half)
                elif wait_b is not None:
                    wb_wait(wait_b, half)
                pltpu.sync_copy(acc_sh.at[pl.ds(r0, _WB)],
                                rows0.at[pl.ds(half * _WB, _WB)])
                pltpu.async_copy(
                    rows0.at[pl.ds(half * _WB, _WB)],
                    out_hbm.at[pl.ds(r0, _WB), pl.ds(chunk * 16, 16)],
                    gsem[half])
                pltpu.sync_copy(rows1.at[pl.ds(0, _WB)],
                                acc_sh.at[pl.ds(r0, _WB)])

            nwb = _RPT // _WB

            @pl.loop(0, nwb // 2)
            def _(j):
                wb_sub(2 * j, 0, 2 * j - 2, j > 0)
                wb_sub(2 * j + 1, 1, 2 * j - 1, j > 0)

            wb_sub(nwb - 1, 0, nwb - 3, None)
            wb_wait(nwb - 2, 1)
            wb_wait(nwb - 1, 0)

            plsc.subcore_barrier()

    return k(hs_flat, src_scaled, dst)


_BR = 2000


def _scale0_body(x_ref, c0_ref, c1_ref, p_ref, nrm_ref):
    deg = c0_ref[0] + c1_ref[0] + 1.0
    nrm = lax.rsqrt(deg)
    nrm_ref[...] = nrm
    p_ref[...] = x_ref[...] * nrm[:, :1]


def _scale0(x16, counts, n):
    g = n // _BR
    return pl.pallas_call(
        _scale0_body,
        grid=(g,),
        in_specs=[
            pl.BlockSpec((_BR, 16), lambda i: (i, 0)),
            pl.BlockSpec((1, _BR, 16), lambda i: (0, i, 0)),
            pl.BlockSpec((1, _BR, 16), lambda i: (1, i, 0)),
        ],
        out_specs=[
            pl.BlockSpec((_BR, 16), lambda i: (i, 0)),
            pl.BlockSpec((_BR, 16), lambda i: (i, 0)),
        ],
        out_shape=[
            jax.ShapeDtypeStruct((n, 16), jnp.float32),
            jax.ShapeDtypeStruct((n, 16), jnp.float32),
        ],
    )(x16, counts, counts)


def _layer1_body(a_ref, p_ref, nrm_ref, b_ref, w_ref, o_ref):
    nrm = nrm_ref[:, :1]
    z = (a_ref[:, :16] + a_ref[:, 16:] + p_ref[...]) * nrm
    h = jnp.dot(z, w_ref[...], preferred_element_type=jnp.float32) + b_ref[...]
    o_ref[...] = jnp.maximum(h, 0.0) * nrm


def _layer1(accp, p, nrm, b1, w1p, n):
    g = n // _BR
    return pl.pallas_call(
        _layer1_body,
        grid=(g,),
        in_specs=[
            pl.BlockSpec((_BR, 32), lambda i: (i, 0)),
            pl.BlockSpec((_BR, 16), lambda i: (i, 0)),
            pl.BlockSpec((_BR, 16), lambda i: (i, 0)),
            pl.BlockSpec((1, 128), lambda i: (0, 0)),
            pl.BlockSpec((16, 128), lambda i: (0, 0)),
        ],
        out_specs=pl.BlockSpec((_BR, 128), lambda i: (i, 0)),
        out_shape=jax.ShapeDtypeStruct((n, 128), jnp.float32),
    )(accp, p, nrm, b1, w1p)


def _layer2_body(a_ref, q_ref, nrm_ref, b_ref, w_ref, o_ref):
    nrm = nrm_ref[:, :1]
    z = (a_ref[...] + q_ref[...]) * nrm
    h = jnp.dot(z, w_ref[...], preferred_element_type=jnp.float32) + b_ref[...]
    o_ref[...] = jnp.maximum(h, 0.0) * nrm


def _layer2(acc, q, nrm, b_prev, w, n):
    g = n // _BR
    f = q.shape[1]
    f2 = w.shape[1]
    return pl.pallas_call(
        _layer2_body,
        grid=(g,),
        in_specs=[
            pl.BlockSpec((_BR, f), lambda i: (i, 0)),
            pl.BlockSpec((_BR, f), lambda i: (i, 0)),
            pl.BlockSpec((_BR, 16), lambda i: (i, 0)),
            pl.BlockSpec((1, f2), lambda i: (0, 0)),
            pl.BlockSpec((f, f2), lambda i: (0, 0)),
        ],
        out_specs=pl.BlockSpec((_BR, f2), lambda i: (i, 0)),
        out_shape=jax.ShapeDtypeStruct((n, f2), jnp.float32),
    )(acc, q, nrm, b_prev, w)


def _head_body(acc_ref, q_ref, nrm_ref, b_ref, w_ref, bat_ref,
               wmu_ref, bmu_ref, wlv_ref, blv_ref,
               mu_ref, lv_ref, psum, cnt, ng):
    i = pl.program_id(0)

    @pl.when(i == 0)
    def _():
        psum[...] = jnp.zeros_like(psum)
        cnt[...] = jnp.zeros_like(cnt)

    nrm = nrm_ref[:, :1]
    z = (acc_ref[...] + q_ref[...]) * nrm
    h = jnp.dot(z, w_ref[...], preferred_element_type=jnp.float32) + b_ref[...]
    h = jnp.maximum(h, 0.0)
    bb = bat_ref[:, :1]
    io = lax.broadcasted_iota(jnp.int32, (_BR, 256), 1)
    oh = jnp.where(bb == io, 1.0, 0.0)
    psum[...] += lax.dot_general(oh, h, (((0,), (0,)), ((), ())),
                                 preferred_element_type=jnp.float32)
    cnt[...] += lax.dot_general(oh, jnp.ones((_BR, 8), jnp.float32),
                                (((0,), (0,)), ((), ())),
                                preferred_element_type=jnp.float32)

    @pl.when(i == ng - 1)
    def _():
        pooled = psum[...] / jnp.maximum(cnt[:, :1], 1.0)
        mu_ref[...] = jnp.dot(pooled, wmu_ref[...],
                              preferred_element_type=jnp.float32) + bmu_ref[...]
        lv_ref[...] = jnp.dot(pooled, wlv_ref[...],
                              preferred_element_type=jnp.float32) + blv_ref[...]


def _head(acc, q, nrm, b3, w3, batchi, wmu, bmu, wlv, blv, n):
    g = n // _BR
    return pl.pallas_call(
        functools.partial(_head_body, ng=g),
        grid=(g,),
        in_specs=[
            pl.BlockSpec((_BR, 256), lambda i: (i, 0)),
            pl.BlockSpec((_BR, 256), lambda i: (i, 0)),
            pl.BlockSpec((_BR, 16), lambda i: (i, 0)),
            pl.BlockSpec((1, 256), lambda i: (0, 0)),
            pl.BlockSpec((256, 256), lambda i: (0, 0)),
            pl.BlockSpec((_BR, 8), lambda i: (i, 0)),
            pl.BlockSpec((256, 64), lambda i: (0, 0)),
            pl.BlockSpec((1, 64), lambda i: (0, 0)),
            pl.BlockSpec((256, 64), lambda i: (0, 0)),
            pl.BlockSpec((1, 64), lambda i: (0, 0)),
        ],
        out_specs=[
            pl.BlockSpec((256, 64), lambda i: (0, 0)),
            pl.BlockSpec((256, 64), lambda i: (0, 0)),
        ],
        out_shape=[
            jax.ShapeDtypeStruct((256, 64), jnp.float32),
            jax.ShapeDtypeStruct((256, 64), jnp.float32),
        ],
        scratch_shapes=[
            pltpu.VMEM((256, 256), jnp.float32),
            pltpu.VMEM((256, 8), jnp.float32),
        ],
    )(acc, q, nrm, b3, w3, batchi, wmu, bmu, wlv, blv)


def kernel(x, edge_index, batch, W1, b1, W2, b2, W3, b3, Wmu, bmu, Wlv, blv):
    n = x.shape[0]
    e = edge_index.shape[1]
    src = edge_index[0]
    dst = edge_index[1]

    counts = _degree_kernel(dst, n, e)

    x16 = jnp.pad(x, ((0, 0), (0, 6)))
    p, nrm = _scale0(x16, counts, n)

    npads = _EPAD - e
    fill = (jnp.arange(npads, dtype=jnp.int32) % 96) + n
    src1p = jnp.concatenate([src, fill])
    dst1p = jnp.concatenate([dst, fill])
    p_pad = jnp.pad(p, ((0, 96), (0, 0)))
    accp = _spmm_kernel(p_pad, src1p, dst1p, 1, _EPAD)

    w1p = jnp.pad(W1, ((0, 6), (0, 0)))
    q1 = _layer1(accp, p, nrm, b1.reshape(1, 128), w1p, n)

    acc1 = _spmm_kernel(q1.reshape(n * 8, 16), src * 8, dst, 8, e)
    q2 = _layer2(acc1, q1, nrm, b2.reshape(1, 256), W2, n)

    acc2 = _spmm_kernel(q2.reshape(n * 16, 16), src * 16, dst, 16, e)

    batchi = jnp.broadcast_to(batch[:, None], (n, 8))
    mu, lv = _head(acc2, q2, nrm, b3.reshape(1, 256), W3,
                   batchi, Wmu, bmu.reshape(1, 64), Wlv, blv.reshape(1, 64), n)
    return (mu, lv)
